# baseline (speedup 1.0000x reference)
import functools

import jax
import jax.numpy as jnp
from jax import lax
from jax.experimental import pallas as pl
from jax.experimental.pallas import tpu as pltpu
from jax.experimental.pallas import tpu_sc as plsc


def _sc_gather(mem, read_idx):
    B = read_idx.shape[0]
    _, D = mem.shape
    info = plsc.get_sparse_core_info()
    NC, NS = info.num_cores, info.num_subcores
    NW = NC * NS
    b_per_w = B // NW
    CH = 128
    n_ch = b_per_w // CH
    mesh = plsc.VectorSubcoreMesh(core_axis_name="c", subcore_axis_name="s")

    @functools.partial(
        pl.kernel,
        mesh=mesh,
        out_type=jax.ShapeDtypeStruct((B, D), jnp.float32),
        scratch_types=[
            pltpu.VMEM((b_per_w,), jnp.int32),
            pltpu.VMEM((b_per_w, D), jnp.float32),
            pltpu.SemaphoreType.DMA,
        ],
        compiler_params=pltpu.CompilerParams(use_tc_tiling_on_sc=False),
    )
    def gather_kernel(read_hbm, table_hbm, out_hbm, idx_v, rows_v, sem):
        wid = lax.axis_index("s") * NC + lax.axis_index("c")
        base = wid * b_per_w
        pltpu.sync_copy(read_hbm.at[pl.ds(base, b_per_w)], idx_v)
        copies = [
            pltpu.make_async_copy(
                table_hbm.at[idx_v.at[pl.ds(t * CH, CH)]],
                rows_v.at[pl.ds(t * CH, CH)],
                sem,
            )
            for t in range(n_ch)
        ]
        for c in copies:
            c.start()
        for c in copies:
            c.wait()
        pltpu.sync_copy(rows_v, out_hbm.at[pl.ds(base, b_per_w)])

    return gather_kernel(read_idx, mem)


def _tc_correction(gathered, idx, val, read_idx):
    B, D = val.shape
    BM, BK = 512, 1024
    grid = (B // BM, B // BK)

    def body(r_ref, c_ref, v_ref, g_ref, o_ref):
        j = pl.program_id(1)
        mask = (r_ref[...] == c_ref[...]).astype(jnp.bfloat16)
        part = jnp.dot(mask, v_ref[...], preferred_element_type=jnp.float32)

        @pl.when(j == 0)
        def _():
            o_ref[...] = g_ref[...] + part

        @pl.when(j > 0)
        def _():
            o_ref[...] += part

    return pl.pallas_call(
        body,
        grid=grid,
        in_specs=[
            pl.BlockSpec((BM, 1), lambda i, j: (i, 0)),
            pl.BlockSpec((1, BK), lambda i, j: (0, j)),
            pl.BlockSpec((BK, D), lambda i, j: (j, 0)),
            pl.BlockSpec((BM, D), lambda i, j: (i, 0)),
        ],
        out_specs=pl.BlockSpec((BM, D), lambda i, j: (i, 0)),
        out_shape=jax.ShapeDtypeStruct((B, D), jnp.float32),
        compiler_params=pltpu.CompilerParams(
            dimension_semantics=("parallel", "arbitrary"),
        ),
    )(read_idx.reshape(B, 1), idx.reshape(1, B), val.astype(jnp.bfloat16), gathered)


def kernel(mem, idx, val, read_idx):
    gathered = _sc_gather(mem, read_idx)
    return _tc_correction(gathered, idx, val, read_idx)
